# baseline (speedup 1.0000x reference)
import functools

import jax
import jax.numpy as jnp
from jax import lax
from jax.experimental import pallas as pl
from jax.experimental.pallas import tpu as pltpu
from jax.experimental.pallas import tpu_sc as plsc

NC = 2
NS = 16
L = 16
NW = NC * NS
B = 16384
D = 64
BPW = B // NW
RING = 32
PF = 6
PFB = 8
NBINS = 320

NU = 1000000
NI = 100000
UW = 26
IW = NW - UW


def _gather_sub(idx_hbm, tbl_hbm, vals_hbm, n_rows, rel_wid, n_role_w,
                idx_v, wl_pk, grp_pk, hist, offs_cur, offs_start,
                blkbuf, rowring, fsem, sem):
    nb = (n_rows + 127) // 128
    bpw = (nb + n_role_w - 1) // n_role_w

    lo = rel_wid * bpw
    iota = jax.lax.iota(jnp.int32, L)
    gdn = lax.GatherDimensionNumbers(
        offset_dims=(), collapsed_slice_dims=(0,), start_index_map=(0,))

    def lane_shuffle(x, perm):
        return lax.gather(x, perm[:, None], dimension_numbers=gdn,
                          slice_sizes=(1,),
                          mode=lax.GatherScatterMode.PROMISE_IN_BOUNDS)

    pltpu.sync_copy(idx_hbm, idx_v)

    zeros = jnp.zeros((L,), jnp.int32)
    for t in range(NBINS // L):
        hist[pl.ds(t * L, L)] = zeros

    def scan_body(v, off):
        idx16 = idx_v[pl.ds(v * L, L)]
        blk = idx16 >> 7
        m = (blk >= lo) & (blk < lo + bpw)
        packed = ((blk - lo) << 21) | ((idx16 & 127) << 14) | (v * L + iota)
        plsc.store_compressed(wl_pk.at[pl.ds(off, L)], packed, mask=m)
        return off + jnp.sum(m.astype(jnp.int32))

    n_w = lax.fori_loop(0, B // L, scan_body, 0)
    nv = (n_w + L - 1) // L

    def hist_body(v, carry):
        valid = (v * L + iota) < n_w
        blkloc = wl_pk[pl.ds(v * L, L)] >> 21
        blkloc = jnp.where(valid, blkloc, NBINS - 1)
        plsc.addupdate_scatter(hist, [blkloc],
                               jnp.ones((L,), jnp.int32), mask=valid)
        return carry

    lax.fori_loop(0, nv, hist_body, 0)

    def pfx_body(t, run):
        h = hist[pl.ds(t * L, L)]
        s = plsc.cumsum(h)
        offs_cur[pl.ds(t * L, L)] = s - h + run
        offs_start[pl.ds(t * L, L)] = s - h + run
        return run + s[L - 1]

    lax.fori_loop(0, NBINS // L, pfx_body, 0)

    def grp_body(v, carry):
        valid = (v * L + iota) < n_w
        w = wl_pk[pl.ds(v * L, L)]
        blkloc = jnp.where(valid, w >> 21, NBINS - 1)
        rank = jnp.zeros((L,), jnp.int32)
        for sh in range(1, L):
            perm = jnp.maximum(iota - sh, 0)
            same = (lane_shuffle(blkloc, perm) == blkloc) & (iota >= sh)
            rank = rank + same.astype(jnp.int32)
        pos = plsc.load_gather(offs_cur, [blkloc]) + rank
        plsc.store_scatter(grp_pk, [pos], w, mask=valid)
        plsc.addupdate_scatter(offs_cur, [blkloc],
                               jnp.ones((L,), jnp.int32), mask=valid)
        return carry

    lax.fori_loop(0, nv, grp_body, 0)

    def ent_body(e, gg_par):
        gg, ppar = gg_par
        w = grp_pk[pl.ds(e, L)][0]
        rl = (w >> 14) & 127
        pid = w & 16383
        slot = gg % RING
        pvec = jnp.full((L,), ppar, jnp.int32)
        cvec = jnp.full((L,), rl, jnp.int32)
        for k in range(D // L):
            rowring[slot, pl.ds(k * L, L)] = plsc.load_gather(
                blkbuf, [pvec, iota + k * L, cvec])
        pltpu.async_copy(
            rowring.at[slot],
            vals_hbm.at[pl.ds(pl.multiple_of(pid * D, 8), D)], sem)

        @pl.when(gg >= RING - 8)
        def _():
            pltpu.make_async_copy(
                rowring.at[0], vals_hbm.at[pl.ds(0, D)], sem).wait()
        return (gg + 1, ppar)

    def fire(bx):
        cnt = hist[pl.ds(bx, L)][0]

        @pl.when((bx < bpw) & (cnt > 0))
        def _():
            col0 = pl.multiple_of((lo + bx) * 128, 128)
            pltpu.async_copy(tbl_hbm.at[:, pl.ds(col0, 128)],
                             blkbuf.at[lax.rem(bx, PFB)], fsem)

    for bx in range(PF):
        fire(jnp.int32(bx))

    def blk_body(bl, g):
        fire(bl + PF)
        par = lax.rem(bl, PFB)
        cnt = hist[pl.ds(bl, L)][0]
        start = offs_start[pl.ds(bl, L)][0]

        @pl.when(cnt > 0)
        def _():
            pltpu.make_async_copy(tbl_hbm.at[:, pl.ds(0, 128)],
                                  blkbuf.at[par], fsem).wait()

        g2, _unused = lax.fori_loop(start, start + cnt, ent_body, (g, par))
        return g2

    g_end = lax.fori_loop(0, bpw, blk_body, 0)

    n_drain = jnp.minimum(g_end, RING - 8)

    def drain_body(i, carry):
        pltpu.make_async_copy(rowring.at[0],
                              vals_hbm.at[pl.ds(0, D)], sem).wait()
        return carry

    lax.fori_loop(0, n_drain, drain_body, 0)


@functools.partial(
    pl.kernel,
    out_type=(jax.ShapeDtypeStruct((B * D,), jnp.float32),
              jax.ShapeDtypeStruct((B * D,), jnp.float32)),
    mesh=plsc.VectorSubcoreMesh(core_axis_name="c", subcore_axis_name="s",
                                num_cores=NC, num_subcores=NS),
    scratch_types=[
        pltpu.VMEM((B,), jnp.int32),
        pltpu.VMEM((B + L,), jnp.int32),
        pltpu.VMEM((B + L,), jnp.int32),
        pltpu.VMEM((NBINS,), jnp.int32),
        pltpu.VMEM((NBINS,), jnp.int32),
        pltpu.VMEM((NBINS,), jnp.int32),
        pltpu.VMEM((PFB, D, 128), jnp.float32),
        pltpu.VMEM((RING, D), jnp.float32),
        pltpu.SemaphoreType.DMA,
        pltpu.SemaphoreType.DMA,
    ],
    compiler_params=pltpu.CompilerParams(needs_layout_passes=False),
)
def _gather_both(uidx, iidx, ufT, ifT, uvals, ivals, *scratch):
    wid = lax.axis_index("s") * NC + lax.axis_index("c")

    @pl.when(wid < UW)
    def _():
        _gather_sub(uidx, ufT, uvals, NU, wid, UW, *scratch)

    @pl.when(wid >= UW)
    def _():
        _gather_sub(iidx, ifT, ivals, NI, wid - UW, IW, *scratch)


def _dot_body(uvals_hbm, ivals_hbm, out_hbm, uv_v, iv_v, out_v, sem):
    wid = lax.axis_index("s") * NC + lax.axis_index("c")
    base = wid * BPW
    pltpu.sync_copy(uvals_hbm.at[pl.ds(base * D, BPW * D)], uv_v)
    pltpu.sync_copy(ivals_hbm.at[pl.ds(base * D, BPW * D)], iv_v)

    iota = jax.lax.iota(jnp.int32, L)
    perms = [iota ^ sh for sh in (8, 4, 2, 1)]
    gdn = lax.GatherDimensionNumbers(offset_dims=(), collapsed_slice_dims=(0,),
                                     start_index_map=(0,))

    def lane_shuffle(x, perm):
        return lax.gather(x, perm[:, None], dimension_numbers=gdn,
                          slice_sizes=(1,),
                          mode=lax.GatherScatterMode.PROMISE_IN_BOUNDS)

    def body(g, carry):
        res = jnp.zeros((L,), jnp.float32)
        for l in range(L):
            p = (g * L + l) * D
            acc = uv_v[pl.ds(p, L)] * iv_v[pl.ds(p, L)]
            for k in range(1, D // L):
                acc = acc + uv_v[pl.ds(p + k * L, L)] * iv_v[pl.ds(p + k * L, L)]
            for perm in perms:
                acc = acc + lane_shuffle(acc, perm)
            res = jnp.where(iota == l, acc, res)
        out_v[pl.ds(g * L, L)] = res
        return carry

    lax.fori_loop(0, BPW // L, body, 0)
    pltpu.sync_copy(out_v, out_hbm.at[pl.ds(base, BPW)])


@functools.partial(
    pl.kernel,
    out_type=jax.ShapeDtypeStruct((B,), jnp.float32),
    mesh=plsc.VectorSubcoreMesh(core_axis_name="c", subcore_axis_name="s",
                                num_cores=NC, num_subcores=NS),
    scratch_types=[
        pltpu.VMEM((BPW * D,), jnp.float32),
        pltpu.VMEM((BPW * D,), jnp.float32),
        pltpu.VMEM((BPW,), jnp.float32),
        pltpu.SemaphoreType.DMA,
    ],
    compiler_params=pltpu.CompilerParams(needs_layout_passes=False),
)
def _dot(uvals, ivals, out, *scratch):
    _dot_body(uvals, ivals, out, *scratch)


def kernel(user, item, user_factors, item_factors):
    ufT = user_factors.T
    ifT = item_factors.T
    uvals, ivals = _gather_both(user.astype(jnp.int32),
                                item.astype(jnp.int32), ufT, ifT)
    return _dot(uvals, ivals)


# speedup vs baseline: 1.4055x; 1.4055x over previous
import functools

import jax
import jax.numpy as jnp
from jax import lax
from jax.experimental import pallas as pl
from jax.experimental.pallas import tpu as pltpu
from jax.experimental.pallas import tpu_sc as plsc

NC = 2
NS = 16
L = 16
NW = NC * NS
B = 16384
D = 64
BPW = B // NW
RING = 32
PF = 6
PFB = 8
NBINS = 272

NU = 1000000
NI = 100000


def _gather_sub(idx_hbm, tbl_hbm, vals_hbm, n_rows,
                idx_v, wl_pk, grp_pk, hist, offs_cur, offs_start,
                blkbuf, rowring, fsem, sem):
    nb = (n_rows + 127) // 128
    bpw = (nb + NW - 1) // NW

    wid = lax.axis_index("s") * NC + lax.axis_index("c")
    lo = wid * bpw
    iota = jax.lax.iota(jnp.int32, L)
    gdn = lax.GatherDimensionNumbers(
        offset_dims=(), collapsed_slice_dims=(0,), start_index_map=(0,))

    def lane_shuffle(x, perm):
        return lax.gather(x, perm[:, None], dimension_numbers=gdn,
                          slice_sizes=(1,),
                          mode=lax.GatherScatterMode.PROMISE_IN_BOUNDS)

    pltpu.sync_copy(idx_hbm, idx_v)

    zeros = jnp.zeros((L,), jnp.int32)
    for t in range(NBINS // L):
        hist[pl.ds(t * L, L)] = zeros

    def scan_body(v, off):
        idx16 = idx_v[pl.ds(v * L, L)]
        blk = idx16 >> 7
        m = (blk >= lo) & (blk < lo + bpw)
        packed = ((blk - lo) << 21) | ((idx16 & 127) << 14) | (v * L + iota)
        plsc.store_compressed(wl_pk.at[pl.ds(off, L)], packed, mask=m)
        return off + jnp.sum(m.astype(jnp.int32))

    n_w = lax.fori_loop(0, B // L, scan_body, 0)
    nv = (n_w + L - 1) // L

    def hist_body(v, carry):
        valid = (v * L + iota) < n_w
        blkloc = wl_pk[pl.ds(v * L, L)] >> 21
        blkloc = jnp.where(valid, blkloc, NBINS - 1)
        plsc.addupdate_scatter(hist, [blkloc],
                               jnp.ones((L,), jnp.int32), mask=valid)
        return carry

    lax.fori_loop(0, nv, hist_body, 0)

    def pfx_body(t, run):
        h = hist[pl.ds(t * L, L)]
        s = plsc.cumsum(h)
        offs_cur[pl.ds(t * L, L)] = s - h + run
        offs_start[pl.ds(t * L, L)] = s - h + run
        return run + s[L - 1]

    lax.fori_loop(0, NBINS // L, pfx_body, 0)

    def grp_body(v, carry):
        valid = (v * L + iota) < n_w
        w = wl_pk[pl.ds(v * L, L)]
        blkloc = jnp.where(valid, w >> 21, NBINS - 1)
        rank = jnp.zeros((L,), jnp.int32)
        for sh in range(1, L):
            perm = jnp.maximum(iota - sh, 0)
            same = (lane_shuffle(blkloc, perm) == blkloc) & (iota >= sh)
            rank = rank + same.astype(jnp.int32)
        pos = plsc.load_gather(offs_cur, [blkloc]) + rank
        plsc.store_scatter(grp_pk, [pos], w, mask=valid)
        plsc.addupdate_scatter(offs_cur, [blkloc],
                               jnp.ones((L,), jnp.int32), mask=valid)
        return carry

    lax.fori_loop(0, nv, grp_body, 0)

    def ent_body(e, gg_par):
        gg, ppar = gg_par
        w = grp_pk[pl.ds(e, L)][0]
        rl = (w >> 14) & 127
        pid = w & 16383
        slot = gg % RING
        pvec = jnp.full((L,), ppar, jnp.int32)
        cvec = jnp.full((L,), rl, jnp.int32)
        for k in range(D // L):
            rowring[slot, pl.ds(k * L, L)] = plsc.load_gather(
                blkbuf, [pvec, iota + k * L, cvec])
        pltpu.async_copy(
            rowring.at[slot],
            vals_hbm.at[pl.ds(pl.multiple_of(pid * D, 8), D)], sem)

        @pl.when(gg >= RING - 8)
        def _():
            pltpu.make_async_copy(
                rowring.at[0], vals_hbm.at[pl.ds(0, D)], sem).wait()
        return (gg + 1, ppar)

    def fire(bx):
        cnt = hist[pl.ds(bx, L)][0]

        @pl.when((bx < bpw) & (cnt > 0))
        def _():
            col0 = pl.multiple_of((lo + bx) * 128, 128)
            pltpu.async_copy(tbl_hbm.at[:, pl.ds(col0, 128)],
                             blkbuf.at[lax.rem(bx, PFB)], fsem)

    for bx in range(PF):
        fire(jnp.int32(bx))

    def blk_body(bl, g):
        fire(bl + PF)
        par = lax.rem(bl, PFB)
        cnt = hist[pl.ds(bl, L)][0]
        start = offs_start[pl.ds(bl, L)][0]

        @pl.when(cnt > 0)
        def _():
            pltpu.make_async_copy(tbl_hbm.at[:, pl.ds(0, 128)],
                                  blkbuf.at[par], fsem).wait()

        g2, _unused = lax.fori_loop(start, start + cnt, ent_body, (g, par))
        return g2

    g_end = lax.fori_loop(0, bpw, blk_body, 0)

    n_drain = jnp.minimum(g_end, RING - 8)

    def drain_body(i, carry):
        pltpu.make_async_copy(rowring.at[0],
                              vals_hbm.at[pl.ds(0, D)], sem).wait()
        return carry

    lax.fori_loop(0, n_drain, drain_body, 0)


@functools.partial(
    pl.kernel,
    out_type=(jax.ShapeDtypeStruct((B * D,), jnp.float32),
              jax.ShapeDtypeStruct((B * D,), jnp.float32)),
    mesh=plsc.VectorSubcoreMesh(core_axis_name="c", subcore_axis_name="s",
                                num_cores=NC, num_subcores=NS),
    scratch_types=[
        pltpu.VMEM((B,), jnp.int32),
        pltpu.VMEM((B + L,), jnp.int32),
        pltpu.VMEM((B + L,), jnp.int32),
        pltpu.VMEM((NBINS,), jnp.int32),
        pltpu.VMEM((NBINS,), jnp.int32),
        pltpu.VMEM((NBINS,), jnp.int32),
        pltpu.VMEM((PFB, D, 128), jnp.float32),
        pltpu.VMEM((RING, D), jnp.float32),
        pltpu.SemaphoreType.DMA,
        pltpu.SemaphoreType.DMA,
    ],
    compiler_params=pltpu.CompilerParams(needs_layout_passes=False),
)
def _gather_both(uidx, iidx, ufT, ifT, uvals, ivals, *scratch):
    _gather_sub(uidx, ufT, uvals, NU, *scratch)
    _gather_sub(iidx, ifT, ivals, NI, *scratch)


def _dot_body(uvals_hbm, ivals_hbm, out_hbm, uv_v, iv_v, out_v, sem):
    wid = lax.axis_index("s") * NC + lax.axis_index("c")
    base = wid * BPW
    pltpu.sync_copy(uvals_hbm.at[pl.ds(base * D, BPW * D)], uv_v)
    pltpu.sync_copy(ivals_hbm.at[pl.ds(base * D, BPW * D)], iv_v)

    iota = jax.lax.iota(jnp.int32, L)
    perms = [iota ^ sh for sh in (8, 4, 2, 1)]
    gdn = lax.GatherDimensionNumbers(offset_dims=(), collapsed_slice_dims=(0,),
                                     start_index_map=(0,))

    def lane_shuffle(x, perm):
        return lax.gather(x, perm[:, None], dimension_numbers=gdn,
                          slice_sizes=(1,),
                          mode=lax.GatherScatterMode.PROMISE_IN_BOUNDS)

    def body(g, carry):
        res = jnp.zeros((L,), jnp.float32)
        for l in range(L):
            p = (g * L + l) * D
            acc = uv_v[pl.ds(p, L)] * iv_v[pl.ds(p, L)]
            for k in range(1, D // L):
                acc = acc + uv_v[pl.ds(p + k * L, L)] * iv_v[pl.ds(p + k * L, L)]
            for perm in perms:
                acc = acc + lane_shuffle(acc, perm)
            res = jnp.where(iota == l, acc, res)
        out_v[pl.ds(g * L, L)] = res
        return carry

    lax.fori_loop(0, BPW // L, body, 0)
    pltpu.sync_copy(out_v, out_hbm.at[pl.ds(base, BPW)])


@functools.partial(
    pl.kernel,
    out_type=jax.ShapeDtypeStruct((B,), jnp.float32),
    mesh=plsc.VectorSubcoreMesh(core_axis_name="c", subcore_axis_name="s",
                                num_cores=NC, num_subcores=NS),
    scratch_types=[
        pltpu.VMEM((BPW * D,), jnp.float32),
        pltpu.VMEM((BPW * D,), jnp.float32),
        pltpu.VMEM((BPW,), jnp.float32),
        pltpu.SemaphoreType.DMA,
    ],
    compiler_params=pltpu.CompilerParams(needs_layout_passes=False),
)
def _dot(uvals, ivals, out, *scratch):
    _dot_body(uvals, ivals, out, *scratch)


def kernel(user, item, user_factors, item_factors):
    ufT = user_factors.T
    ifT = item_factors.T
    uvals, ivals = _gather_both(user.astype(jnp.int32),
                                item.astype(jnp.int32), ufT, ifT)
    return _dot(uvals, ivals)
